# baseline (speedup 1.0000x reference)
import functools

import jax
import jax.numpy as jnp
from jax import lax
from jax.experimental import pallas as pl
from jax.experimental.pallas import tpu as pltpu
from jax.experimental.pallas import tpu_sc as plsc

DIM = 64
ROWS = 100
SB = 16


@functools.lru_cache(maxsize=None)
def _build(num_blocks):
    info = plsc.get_sparse_core_info()
    nc, ns = info.num_cores, info.num_subcores
    nw = nc * ns
    bpw = num_blocks // nw
    nsb = bpw // SB
    mesh = plsc.VectorSubcoreMesh(core_axis_name="c", subcore_axis_name="s")

    @functools.partial(
        pl.kernel,
        mesh=mesh,
        out_type=jax.ShapeDtypeStruct((num_blocks, ROWS, DIM), jnp.float32),
        compiler_params=pltpu.CompilerParams(use_tc_tiling_on_sc=False),
        scratch_types=[
            pltpu.VMEM((SB, ROWS), jnp.int32),
            pltpu.VMEM((SB, ROWS), jnp.int32),
            pltpu.VMEM((2, ROWS), jnp.int32),
            pltpu.VMEM((2, ROWS, DIM), jnp.float32),
            pltpu.VMEM((2, ROWS, DIM), jnp.float32),
            pltpu.VMEM_SHARED((ns * 2 * ROWS, DIM), jnp.float32),
            pltpu.SemaphoreType.DMA,
            pltpu.SemaphoreType.DMA,
            pltpu.SemaphoreType.DMA,
            pltpu.SemaphoreType.DMA,
        ],
    )
    def k(ex_idx, comb_idx, ident, ex_tab, comb_tab, out,
          exidx_s, combidx_s, identv, exbuf, combbuf, slab,
          sem_g0, sem_g1, sem_w0, sem_w1):
        cid = lax.axis_index("c")
        sid = lax.axis_index("s")
        wid = sid * nc + cid
        base = wid * bpw
        sem_g = (sem_g0, sem_g1)
        sem_w = (sem_w0, sem_w1)
        pltpu.sync_copy(ident.at[sid], identv)

        def slab_slice(c):
            return slab.at[pl.ds((sid * 2 + c) * ROWS, ROWS)]

        def issue_gathers(i, j, c):
            pltpu.async_copy(ex_tab.at[exidx_s.at[j]], exbuf.at[c], sem_g[c])
            pltpu.async_copy(comb_tab.at[combidx_s.at[j]], combbuf.at[c],
                             sem_g[c])

        def consume(i, j, c):
            pltpu.make_async_copy(
                ex_tab.at[exidx_s.at[j]], exbuf.at[c], sem_g[c]).wait()
            pltpu.make_async_copy(
                comb_tab.at[combidx_s.at[j]], combbuf.at[c], sem_g[c]).wait()

            @pl.when(i >= 2)
            def _():
                pltpu.make_async_copy(
                    slab_slice(c), out.at[base + i], sem_w[c]).wait()

            pltpu.sync_copy(exbuf.at[c], slab_slice(c))
            pltpu.sync_copy(combbuf.at[c], slab.at[identv.at[c]], add=True)
            pltpu.async_copy(slab_slice(c), out.at[base + i], sem_w[c])

        def outer(sb, carry):
            sb_base = base + sb * SB
            pltpu.sync_copy(ex_idx.at[pl.ds(sb_base, SB)], exidx_s)
            pltpu.sync_copy(comb_idx.at[pl.ds(sb_base, SB)], combidx_s)
            issue_gathers(sb * SB, 0, 0)

            def inner(g, carry2):
                for half in range(2):
                    j = 2 * g + half
                    c = half
                    i = sb * SB + j

                    @pl.when(j + 1 < SB)
                    def _():
                        issue_gathers(i + 1, j + 1, 1 - c)

                    consume(i, j, c)
                return carry2

            lax.fori_loop(0, SB // 2, inner, 0)
            return carry

        lax.fori_loop(0, nsb, outer, 0)
        for c in range(2):
            pltpu.make_async_copy(
                slab_slice(c), out.at[base], sem_w[c]).wait()

    return k


def kernel(exercise_tokens, category_tokens, exercise_table, category_table,
           position_table):
    batch, seq = exercise_tokens.shape
    dim = exercise_table.shape[1]
    num_blocks = (batch * seq) // ROWS
    ex_idx = exercise_tokens.reshape(num_blocks, ROWS).astype(jnp.int32)
    comb_idx = (category_tokens.astype(jnp.int32) * seq
                + jnp.arange(seq, dtype=jnp.int32)[None, :])
    comb_idx = comb_idx.reshape(num_blocks, ROWS)
    comb_tab = (category_table[:, None, :]
                + position_table[None, :, :]).reshape(-1, dim)
    info = plsc.get_sparse_core_info()
    ident = (jnp.arange(ROWS, dtype=jnp.int32)[None, :]
             + ROWS * jnp.arange(info.num_subcores * 2,
                                 dtype=jnp.int32)[:, None])
    ident = ident.reshape(info.num_subcores, 2, ROWS)
    k = _build(num_blocks)
    out = k(ex_idx, comb_idx, ident, exercise_table, comb_tab)
    return out.reshape(batch, seq, dim)


# speedup vs baseline: 1.0008x; 1.0008x over previous
import functools

import jax
import jax.numpy as jnp
from jax import lax
from jax.experimental import pallas as pl
from jax.experimental.pallas import tpu as pltpu
from jax.experimental.pallas import tpu_sc as plsc

DIM = 64
ROWS = 100
SB = 16


@functools.lru_cache(maxsize=None)
def _build(num_blocks):
    info = plsc.get_sparse_core_info()
    nc, ns = info.num_cores, info.num_subcores
    nw = nc * ns
    bpw = num_blocks // nw
    nsb = bpw // SB
    mesh = plsc.VectorSubcoreMesh(core_axis_name="c", subcore_axis_name="s")

    @functools.partial(
        pl.kernel,
        mesh=mesh,
        out_type=jax.ShapeDtypeStruct((num_blocks // 2, 2 * ROWS, DIM),
                                      jnp.float32),
        compiler_params=pltpu.CompilerParams(use_tc_tiling_on_sc=False),
        scratch_types=[
            pltpu.VMEM((SB, ROWS), jnp.int32),
            pltpu.VMEM((SB, ROWS), jnp.int32),
            pltpu.VMEM((2, ROWS), jnp.int32),
            pltpu.VMEM((2, ROWS, DIM), jnp.float32),
            pltpu.VMEM((2, ROWS, DIM), jnp.float32),
            pltpu.VMEM_SHARED((ns * 2 * ROWS, DIM), jnp.float32),
            pltpu.SemaphoreType.DMA,
            pltpu.SemaphoreType.DMA,
            pltpu.SemaphoreType.DMA,
            pltpu.SemaphoreType.DMA,
        ],
    )
    def k(ex_idx, comb_idx, ident, ex_tab, comb_tab, out,
          exidx_s, combidx_s, identv, exbuf, combbuf, slab,
          sem_g0, sem_g1, sem_w0, sem_w1):
        cid = lax.axis_index("c")
        sid = lax.axis_index("s")
        wid = sid * nc + cid
        base = wid * bpw
        sem_g = (sem_g0, sem_g1)
        sem_w = (sem_w0, sem_w1)
        pltpu.sync_copy(ident.at[sid], identv)

        def slab_slice(c):
            return slab.at[pl.ds((sid * 2 + c) * ROWS, ROWS)]

        def out_slice(i):
            r = base + i
            return out.at[lax.div(r, 2)].at[pl.ds(lax.rem(r, 2) * ROWS, ROWS)]

        def issue_gathers(i, j, c):
            pltpu.async_copy(ex_tab.at[exidx_s.at[j]], exbuf.at[c], sem_g[c])
            pltpu.async_copy(comb_tab.at[combidx_s.at[j]], combbuf.at[c],
                             sem_g[c])

        def consume(i, j, c):
            pltpu.make_async_copy(
                ex_tab.at[exidx_s.at[j]], exbuf.at[c], sem_g[c]).wait()
            pltpu.make_async_copy(
                comb_tab.at[combidx_s.at[j]], combbuf.at[c], sem_g[c]).wait()

            @pl.when(i >= 2)
            def _():
                pltpu.make_async_copy(
                    slab_slice(c), out_slice(i), sem_w[c]).wait()

            pltpu.sync_copy(exbuf.at[c], slab_slice(c))
            pltpu.sync_copy(combbuf.at[c], slab.at[identv.at[c]], add=True)
            pltpu.async_copy(slab_slice(c), out_slice(i), sem_w[c])

        def outer(sb, carry):
            sb_base = base + sb * SB
            pltpu.sync_copy(ex_idx.at[pl.ds(sb_base, SB)], exidx_s)
            pltpu.sync_copy(comb_idx.at[pl.ds(sb_base, SB)], combidx_s)
            issue_gathers(sb * SB, 0, 0)

            def inner(g, carry2):
                for half in range(2):
                    j = 2 * g + half
                    c = half
                    i = sb * SB + j

                    @pl.when(j + 1 < SB)
                    def _():
                        issue_gathers(i + 1, j + 1, 1 - c)

                    consume(i, j, c)
                return carry2

            lax.fori_loop(0, SB // 2, inner, 0)
            return carry

        lax.fori_loop(0, nsb, outer, 0)
        for c in range(2):
            pltpu.make_async_copy(
                slab_slice(c), out_slice(c), sem_w[c]).wait()

    return k


def kernel(exercise_tokens, category_tokens, exercise_table, category_table,
           position_table):
    batch, seq = exercise_tokens.shape
    dim = exercise_table.shape[1]
    num_blocks = (batch * seq) // ROWS
    ex_idx = exercise_tokens.reshape(num_blocks, ROWS).astype(jnp.int32)
    comb_idx = (category_tokens.astype(jnp.int32) * seq
                + jnp.arange(seq, dtype=jnp.int32)[None, :])
    comb_idx = comb_idx.reshape(num_blocks, ROWS)
    comb_tab = (category_table[:, None, :]
                + position_table[None, :, :]).reshape(-1, dim)
    info = plsc.get_sparse_core_info()
    ident = (jnp.arange(ROWS, dtype=jnp.int32)[None, :]
             + ROWS * jnp.arange(info.num_subcores * 2,
                                 dtype=jnp.int32)[:, None])
    ident = ident.reshape(info.num_subcores, 2, ROWS)
    k = _build(num_blocks)
    out = k(ex_idx, comb_idx, ident, exercise_table, comb_tab)
    return out.reshape(batch, seq, dim)


# speedup vs baseline: 1.2367x; 1.2357x over previous
import functools

import jax
import jax.numpy as jnp
from jax import lax
from jax.experimental import pallas as pl
from jax.experimental.pallas import tpu as pltpu
from jax.experimental.pallas import tpu_sc as plsc

DIM = 64
ROWS = 100
SB = 16


@functools.lru_cache(maxsize=None)
def _build(num_blocks):
    info = plsc.get_sparse_core_info()
    nc, ns = info.num_cores, info.num_subcores
    nw = nc * ns
    bpw = num_blocks // nw
    nsb = bpw // SB
    mesh = plsc.VectorSubcoreMesh(core_axis_name="c", subcore_axis_name="s")

    @functools.partial(
        pl.kernel,
        mesh=mesh,
        out_type=jax.ShapeDtypeStruct((num_blocks * ROWS, DIM), jnp.float32),
        compiler_params=pltpu.CompilerParams(use_tc_tiling_on_sc=False),
        scratch_types=[
            pltpu.VMEM((SB, ROWS), jnp.int32),
            pltpu.VMEM((SB, ROWS), jnp.int32),
            pltpu.VMEM((SB, ROWS), jnp.int32),
            pltpu.VMEM((2, ROWS), jnp.int32),
            pltpu.VMEM((2, ROWS, DIM), jnp.float32),
            pltpu.VMEM((2, ROWS, DIM), jnp.float32),
            pltpu.VMEM((2, ROWS, DIM), jnp.float32),
            pltpu.VMEM_SHARED((ns * 2 * ROWS, DIM), jnp.float32),
            pltpu.SemaphoreType.DMA,
            pltpu.SemaphoreType.DMA,
            pltpu.SemaphoreType.DMA,
            pltpu.SemaphoreType.DMA,
        ],
    )
    def k(ex_idx, comb_idx, scat_idx, ident, ex_tab, comb_tab, out,
          exidx_s, combidx_s, scatidx_s, identv, exbuf, combbuf, outbuf, slab,
          sem_g0, sem_g1, sem_w0, sem_w1):
        cid = lax.axis_index("c")
        sid = lax.axis_index("s")
        wid = sid * nc + cid
        base = wid * bpw
        sem_g = (sem_g0, sem_g1)
        sem_w = (sem_w0, sem_w1)
        pltpu.sync_copy(ident.at[sid], identv)

        def slab_slice(c):
            return slab.at[pl.ds((sid * 2 + c) * ROWS, ROWS)]

        def issue_gathers(j, c):
            pltpu.async_copy(ex_tab.at[exidx_s.at[j]], exbuf.at[c], sem_g[c])
            pltpu.async_copy(comb_tab.at[combidx_s.at[j]], combbuf.at[c],
                             sem_g[c])

        def consume(i, j, c):
            pltpu.make_async_copy(
                ex_tab.at[exidx_s.at[j]], exbuf.at[c], sem_g[c]).wait()
            pltpu.make_async_copy(
                comb_tab.at[combidx_s.at[j]], combbuf.at[c], sem_g[c]).wait()

            @pl.when(i >= 2)
            def _():
                pltpu.make_async_copy(
                    outbuf.at[c], out.at[scatidx_s.at[j]], sem_w[c]).wait()

            pltpu.sync_copy(exbuf.at[c], slab_slice(c))
            pltpu.sync_copy(combbuf.at[c], slab.at[identv.at[c]], add=True)
            pltpu.sync_copy(slab_slice(c), outbuf.at[c])
            pltpu.async_copy(outbuf.at[c], out.at[scatidx_s.at[j]], sem_w[c])

        def outer(sb, carry):
            sb_base = base + sb * SB
            pltpu.sync_copy(ex_idx.at[pl.ds(sb_base, SB)], exidx_s)
            pltpu.sync_copy(comb_idx.at[pl.ds(sb_base, SB)], combidx_s)
            pltpu.sync_copy(scat_idx.at[pl.ds(sb_base, SB)], scatidx_s)
            issue_gathers(0, 0)

            def inner(g, carry2):
                for half in range(2):
                    j = 2 * g + half
                    c = half
                    i = sb * SB + j

                    @pl.when(j + 1 < SB)
                    def _():
                        issue_gathers(j + 1, 1 - c)

                    consume(i, j, c)
                return carry2

            lax.fori_loop(0, SB // 2, inner, 0)
            return carry

        lax.fori_loop(0, nsb, outer, 0)
        for c in range(2):
            pltpu.make_async_copy(
                outbuf.at[c], out.at[scatidx_s.at[SB - 2 + c]],
                sem_w[c]).wait()

    return k


def _transpose_tc(z2, seq, batch):
    def body(z_ref, x_ref):
        x = z_ref[...]
        x_ref[...] = x.T.reshape(2, DIM, batch)

    return pl.pallas_call(
        body,
        grid=(seq // 2,),
        in_specs=[pl.BlockSpec((batch, 2 * DIM), lambda i: (i, 0))],
        out_specs=pl.BlockSpec((2, DIM, batch), lambda i: (i, 0, 0)),
        out_shape=jax.ShapeDtypeStruct((seq, DIM, batch), jnp.float32),
    )(z2)


def kernel(exercise_tokens, category_tokens, exercise_table, category_table,
           position_table):
    batch, seq = exercise_tokens.shape
    dim = exercise_table.shape[1]
    num_blocks = (batch * seq) // ROWS
    ex_idx = exercise_tokens.reshape(num_blocks, ROWS).astype(jnp.int32)
    comb_idx = (category_tokens.astype(jnp.int32) * seq
                + jnp.arange(seq, dtype=jnp.int32)[None, :])
    comb_idx = comb_idx.reshape(num_blocks, ROWS)
    comb_tab = (category_table[:, None, :]
                + position_table[None, :, :]).reshape(-1, dim)
    ls = jnp.arange(seq, dtype=jnp.int32)[None, :]
    bs = jnp.arange(batch, dtype=jnp.int32)[:, None]
    scat_idx = ((ls // 2) * (2 * batch) + bs * 2 + (ls % 2))
    scat_idx = scat_idx.reshape(num_blocks, ROWS)
    info = plsc.get_sparse_core_info()
    ident = (jnp.arange(ROWS, dtype=jnp.int32)[None, :]
             + ROWS * jnp.arange(info.num_subcores * 2,
                                 dtype=jnp.int32)[:, None])
    ident = ident.reshape(info.num_subcores, 2, ROWS)
    k = _build(num_blocks)
    z = k(ex_idx, comb_idx, scat_idx, ident, exercise_table, comb_tab)
    z2 = z.reshape((batch * seq) // 2, 2 * dim)
    x = _transpose_tc(z2, seq, batch)
    return jnp.transpose(x, (2, 0, 1))
